# baseline (speedup 1.0000x reference)
import functools

import jax
import jax.numpy as jnp
from jax.experimental import pallas as pl
from jax.experimental.pallas import tpu as pltpu

_IMG = 224.0
_CROP = 7.0
_ROI = 2
_A = _ROI / _CROP
_NCHUNK = 4
_NSPLIT = 1


def _axis_weight_mats(coord_row, dimsize, L):
    lmn = coord_row / _IMG * _CROP
    t = -1.0 + 2.0 * lmn / _CROP
    iota = jax.lax.broadcasted_iota(jnp.int32, (dimsize, L), 0)
    mats = []
    for p in range(_ROI):
        base = (2.0 * p + 1.0) / _ROI - 1.0
        g = _A * base + t
        pos = jnp.clip(((g + 1.0) * dimsize - 1.0) * 0.5, 0.0, dimsize - 1.0)
        p0f = jnp.floor(pos)
        frac = pos - p0f
        p0 = p0f.astype(jnp.int32)
        p1 = jnp.minimum(p0 + 1, dimsize - 1)
        m = jnp.where(iota == p0, 1.0 - frac, 0.0) + jnp.where(
            iota == p1, frac, 0.0
        )
        mats.append(m)
    return mats


def _pool_kernel(lmx_ref, lmy_ref, *rest, C, H, W, L):
    feat_refs = rest[:_NCHUNK]
    out_ref = rest[_NCHUNK]
    cc = C // _NCHUNK

    lx = lmx_ref[0]
    ly = lmy_ref[0]

    Wx0, Wx1 = _axis_weight_mats(lx, W, L)
    Wy0, Wy1 = _axis_weight_mats(ly, H, L)

    Wx = jnp.concatenate([Wx0, Wx1], axis=1).astype(jnp.bfloat16)
    Wy0r = jnp.concatenate([Wy0, Wy0], axis=1)[None]
    Wy1r = jnp.concatenate([Wy1, Wy1], axis=1)[None]

    vms = []
    for k in range(_NCHUNK):
        tmp = jnp.dot(
            feat_refs[k][0], Wx, preferred_element_type=jnp.float32
        )
        tmp3 = tmp.reshape(cc, H, 2 * L)
        v0 = jnp.sum(tmp3 * Wy0r, axis=1)
        v1 = jnp.sum(tmp3 * Wy1r, axis=1)
        vms.append(jnp.maximum(v0, v1))
    vm = jnp.concatenate(vms, axis=0)
    pooled = jnp.maximum(vm[:, :L], vm[:, L:])
    out_ref[0] = pooled.T


def _mm_kernel(x_ref, w_ref, b_ref, out_ref):
    acc = jax.lax.dot_general(
        x_ref[...],
        w_ref[...],
        (((1,), (1,)), ((), ())),
        preferred_element_type=jnp.float32,
    )
    out_ref[...] = jnp.maximum(acc + b_ref[...], 0.0)


def kernel(features, landmarks, W_lin, b_lin):
    B, C, H, W = features.shape
    L = landmarks.shape[1] // 2
    OUT, K = W_lin.shape
    cc = C // _NCHUNK

    lmx = landmarks[:, 0::2].reshape(B, 1, L)
    lmy = landmarks[:, 1::2].reshape(B, 1, L)

    chunk_specs = [
        pl.BlockSpec((1, cc * H, W), functools.partial(
            lambda b, kk: (b, kk, 0), kk=k))
        for k in range(_NCHUNK)
    ]

    bs = B // _NSPLIT
    pooled_parts = []
    for part in range(_NSPLIT):
        sl = slice(part * bs, (part + 1) * bs)
        featCH = features[sl].astype(jnp.bfloat16).reshape(bs, C * H, W)
        pooled_parts.append(pl.pallas_call(
            functools.partial(_pool_kernel, C=C, H=H, W=W, L=L),
            grid=(bs,),
            in_specs=[
                pl.BlockSpec((1, 1, L), lambda b: (b, 0, 0)),
                pl.BlockSpec((1, 1, L), lambda b: (b, 0, 0)),
            ] + chunk_specs,
            out_specs=pl.BlockSpec((1, L, C), lambda b: (b, 0, 0)),
            out_shape=jax.ShapeDtypeStruct((bs, L, C), jnp.float32),
            compiler_params=pltpu.CompilerParams(
                dimension_semantics=("parallel",),
            ),
            name="roi_pool",
        )(lmx[sl], lmy[sl], *([featCH] * _NCHUNK)))
    pooled = jnp.concatenate(pooled_parts, axis=0)

    flat = pooled.reshape(B, L * C)
    NB = 512
    b2 = b_lin.reshape(1, OUT)
    out = pl.pallas_call(
        _mm_kernel,
        grid=(OUT // NB,),
        in_specs=[
            pl.BlockSpec((B, K), lambda i: (0, 0)),
            pl.BlockSpec((NB, K), lambda i: (i, 0)),
            pl.BlockSpec((1, NB), lambda i: (0, i)),
        ],
        out_specs=pl.BlockSpec((B, NB), lambda i: (0, i)),
        out_shape=jax.ShapeDtypeStruct((B, OUT), jnp.float32),
        compiler_params=pltpu.CompilerParams(
            dimension_semantics=("parallel",),
        ),
        name="linear_relu",
    )(flat, W_lin, b2)
    return out


# speedup vs baseline: 1.1956x; 1.1956x over previous
import functools

import jax
import jax.numpy as jnp
from jax.experimental import pallas as pl
from jax.experimental.pallas import tpu as pltpu

_IMG = 224.0
_CROP = 7.0
_ROI = 2
_A = _ROI / _CROP
_NCHUNK = 4
_NSPLIT = 1


def _axis_weight_mats(coord_row, dimsize, L):
    lmn = coord_row / _IMG * _CROP
    t = -1.0 + 2.0 * lmn / _CROP
    iota = jax.lax.broadcasted_iota(jnp.int32, (dimsize, L), 0)
    mats = []
    for p in range(_ROI):
        base = (2.0 * p + 1.0) / _ROI - 1.0
        g = _A * base + t
        pos = jnp.clip(((g + 1.0) * dimsize - 1.0) * 0.5, 0.0, dimsize - 1.0)
        p0f = jnp.floor(pos)
        frac = pos - p0f
        p0 = p0f.astype(jnp.int32)
        p1 = jnp.minimum(p0 + 1, dimsize - 1)
        m = jnp.where(iota == p0, 1.0 - frac, 0.0) + jnp.where(
            iota == p1, frac, 0.0
        )
        mats.append(m)
    return mats


def _pool_kernel(lmx_ref, lmy_ref, *rest, C, H, W, L):
    feat_refs = rest[:_NCHUNK]
    out_ref = rest[_NCHUNK]
    cc = C // _NCHUNK

    lx = lmx_ref[0]
    ly = lmy_ref[0]

    Wx0, Wx1 = _axis_weight_mats(lx, W, L)
    Wy0, Wy1 = _axis_weight_mats(ly, H, L)

    Wx = jnp.concatenate([Wx0, Wx1], axis=1)
    Wy0r = jnp.concatenate([Wy0, Wy0], axis=1)[None]
    Wy1r = jnp.concatenate([Wy1, Wy1], axis=1)[None]

    vms = []
    for k in range(_NCHUNK):
        tmp = jnp.dot(
            feat_refs[k][0], Wx, preferred_element_type=jnp.float32
        )
        tmp3 = tmp.reshape(cc, H, 2 * L)
        v0 = jnp.sum(tmp3 * Wy0r, axis=1)
        v1 = jnp.sum(tmp3 * Wy1r, axis=1)
        vms.append(jnp.maximum(v0, v1))
    vm = jnp.concatenate(vms, axis=0)
    pooled = jnp.maximum(vm[:, :L], vm[:, L:])
    out_ref[0] = pooled.T


def _mm_kernel(x_ref, w_ref, b_ref, out_ref):
    acc = jax.lax.dot_general(
        x_ref[...],
        w_ref[...],
        (((1,), (1,)), ((), ())),
        preferred_element_type=jnp.float32,
    )
    out_ref[...] = jnp.maximum(acc + b_ref[...], 0.0)


def kernel(features, landmarks, W_lin, b_lin):
    B, C, H, W = features.shape
    L = landmarks.shape[1] // 2
    OUT, K = W_lin.shape
    cc = C // _NCHUNK

    lmx = landmarks[:, 0::2].reshape(B, 1, L)
    lmy = landmarks[:, 1::2].reshape(B, 1, L)

    chunk_specs = [
        pl.BlockSpec((1, cc * H, W), functools.partial(
            lambda b, kk: (b, kk, 0), kk=k))
        for k in range(_NCHUNK)
    ]

    bs = B // _NSPLIT
    pooled_parts = []
    for part in range(_NSPLIT):
        sl = slice(part * bs, (part + 1) * bs)
        featCH = features[sl].reshape(bs, C * H, W)
        pooled_parts.append(pl.pallas_call(
            functools.partial(_pool_kernel, C=C, H=H, W=W, L=L),
            grid=(bs,),
            in_specs=[
                pl.BlockSpec((1, 1, L), lambda b: (b, 0, 0)),
                pl.BlockSpec((1, 1, L), lambda b: (b, 0, 0)),
            ] + chunk_specs,
            out_specs=pl.BlockSpec((1, L, C), lambda b: (b, 0, 0)),
            out_shape=jax.ShapeDtypeStruct((bs, L, C), jnp.float32),
            compiler_params=pltpu.CompilerParams(
                dimension_semantics=("parallel",),
            ),
            name="roi_pool",
        )(lmx[sl], lmy[sl], *([featCH] * _NCHUNK)))
    pooled = jnp.concatenate(pooled_parts, axis=0)

    flat = pooled.reshape(B, L * C)
    NB = 512
    b2 = b_lin.reshape(1, OUT)
    out = pl.pallas_call(
        _mm_kernel,
        grid=(OUT // NB,),
        in_specs=[
            pl.BlockSpec((B, K), lambda i: (0, 0)),
            pl.BlockSpec((NB, K), lambda i: (i, 0)),
            pl.BlockSpec((1, NB), lambda i: (0, i)),
        ],
        out_specs=pl.BlockSpec((B, NB), lambda i: (0, i)),
        out_shape=jax.ShapeDtypeStruct((B, OUT), jnp.float32),
        compiler_params=pltpu.CompilerParams(
            dimension_semantics=("parallel",),
        ),
        name="linear_relu",
    )(flat, W_lin, b2)
    return out


# speedup vs baseline: 4.8201x; 4.0316x over previous
import functools

import jax
import jax.numpy as jnp
from jax.experimental import pallas as pl
from jax.experimental.pallas import tpu as pltpu

_IMG = 224.0
_CROP = 7.0
_ROI = 2
_A = _ROI / _CROP

_WROWS = 24
_NW = 4


def _axis_params(coord, dimsize):
    lmn = coord / _IMG * _CROP
    t = -1.0 + 2.0 * lmn / _CROP
    out = []
    for p in range(_ROI):
        base = (2.0 * p + 1.0) / _ROI - 1.0
        g = _A * base + t
        pos = jnp.clip(((g + 1.0) * dimsize - 1.0) * 0.5, 0.0, dimsize - 1.0)
        i0f = jnp.floor(pos)
        f = pos - i0f
        i0 = i0f.astype(jnp.int32)
        i1 = jnp.minimum(i0 + 1, dimsize - 1)
        out.append((i0, i1, f))
    return out


def _pool_kernel(offs, t00, t01, t10, t11, w00, w01, w10, w11,
                 feat, out_ref, scratch, sems, *, C, L, nb_per_core):
    core = pl.program_id(0)
    j = pl.program_id(1)
    nslot = _NW * L

    def issue(step_in_core, buf):
        b = core * nb_per_core + step_in_core
        for k in range(nslot):
            ws = pl.multiple_of(offs[b, k], 8)
            pltpu.make_async_copy(
                feat.at[b, pl.ds(ws, _WROWS), :],
                scratch.at[buf, pl.ds(k * _WROWS, _WROWS), :],
                sems.at[buf],
            ).start()

    @pl.when(j == 0)
    def _():
        issue(j, 0)

    @pl.when(j + 1 < nb_per_core)
    def _():
        issue(j + 1, (j + 1) % 2)

    cur = j % 2
    pltpu.make_async_copy(
        scratch.at[cur], scratch.at[cur], sems.at[cur]
    ).wait()

    iota_m = jax.lax.broadcasted_iota(jnp.int32, (4 * L, nslot * _WROWS), 1)
    coef = (
        jnp.where(iota_m == t00[0], w00[0], 0.0)
        + jnp.where(iota_m == t01[0], w01[0], 0.0)
        + jnp.where(iota_m == t10[0], w10[0], 0.0)
        + jnp.where(iota_m == t11[0], w11[0], 0.0)
    )

    vals = jnp.dot(
        coef, scratch[cur], preferred_element_type=jnp.float32
    )
    pooled = jnp.max(vals.reshape(L, 4, C), axis=1)
    out_ref[0] = pooled


def _mm_kernel(x_ref, w_ref, b_ref, out_ref):
    acc = jax.lax.dot_general(
        x_ref[...],
        w_ref[...],
        (((1,), (1,)), ((), ())),
        preferred_element_type=jnp.float32,
    )
    out_ref[...] = jnp.maximum(acc + b_ref[...], 0.0)


def kernel(features, landmarks, W_lin, b_lin):
    B, C, H, W = features.shape
    L = landmarks.shape[1] // 2
    OUT, K = W_lin.shape
    ncores = 2
    nb_per_core = B // ncores
    HW = H * W

    featT = features.transpose(0, 2, 3, 1).reshape(B, HW, C)

    lmx = landmarks[:, 0::2]
    lmy = landmarks[:, 1::2]
    xp = _axis_params(lmx, W)
    yp = _axis_params(lmy, H)

    yrows = [yp[0][0], yp[0][1], yp[1][0], yp[1][1]]
    xbase = xp[0][0]

    ws_list, off_list = [], []
    for yr in yrows:
        r = yr * W + xbase
        ws = jnp.minimum((r // 8) * 8, HW - _WROWS)
        ws_list.append(ws)
    offs = jnp.stack(ws_list, axis=-1).reshape(B, _NW * L).astype(jnp.int32)

    def per_q(fn):
        cols = [fn(py, px) for py in range(2) for px in range(2)]
        return jnp.stack(cols, axis=-1).reshape(B, L * 4, 1)

    lidx = jnp.arange(L, dtype=jnp.int32)[None, :]

    def tgt(py, px, yc, xc):
        k = lidx * 4 + 2 * py + yc
        y = yp[py][yc]
        x = xp[px][xc]
        ws = ws_list[2 * py + yc]
        return k * _WROWS + (y * W + x - ws)

    q_t00 = per_q(lambda py, px: tgt(py, px, 0, 0)).astype(jnp.int32)
    q_t01 = per_q(lambda py, px: tgt(py, px, 0, 1)).astype(jnp.int32)
    q_t10 = per_q(lambda py, px: tgt(py, px, 1, 0)).astype(jnp.int32)
    q_t11 = per_q(lambda py, px: tgt(py, px, 1, 1)).astype(jnp.int32)
    q_w00 = per_q(lambda py, px: (1.0 - yp[py][2]) * (1.0 - xp[px][2]))
    q_w01 = per_q(lambda py, px: (1.0 - yp[py][2]) * xp[px][2])
    q_w10 = per_q(lambda py, px: yp[py][2] * (1.0 - xp[px][2]))
    q_w11 = per_q(lambda py, px: yp[py][2] * xp[px][2])

    lane_params = [q_t00, q_t01, q_t10, q_t11, q_w00, q_w01, q_w10, q_w11]
    lane_specs = [
        pl.BlockSpec((1, 4 * L, 1),
                     lambda c, j, *refs: (c * nb_per_core + j, 0, 0))
        for _ in lane_params
    ]

    pooled = pl.pallas_call(
        functools.partial(_pool_kernel, C=C, L=L, nb_per_core=nb_per_core),
        grid_spec=pltpu.PrefetchScalarGridSpec(
            num_scalar_prefetch=1,
            grid=(ncores, nb_per_core),
            in_specs=lane_specs + [pl.BlockSpec(memory_space=pl.ANY)],
            out_specs=pl.BlockSpec(
                (1, L, C),
                lambda c, j, *refs: (c * nb_per_core + j, 0, 0),
            ),
            scratch_shapes=[
                pltpu.VMEM((2, _NW * L * _WROWS, C), jnp.float32),
                pltpu.SemaphoreType.DMA((2,)),
            ],
        ),
        out_shape=jax.ShapeDtypeStruct((B, L, C), jnp.float32),
        compiler_params=pltpu.CompilerParams(
            dimension_semantics=("parallel", "arbitrary"),
        ),
        name="roi_pool_gather",
    )(offs, *lane_params, featT)

    flat = pooled.reshape(B, L * C)
    NB = 512
    b2 = b_lin.reshape(1, OUT)
    out = pl.pallas_call(
        _mm_kernel,
        grid=(OUT // NB,),
        in_specs=[
            pl.BlockSpec((B, K), lambda i: (0, 0)),
            pl.BlockSpec((NB, K), lambda i: (i, 0)),
            pl.BlockSpec((1, NB), lambda i: (0, i)),
        ],
        out_specs=pl.BlockSpec((B, NB), lambda i: (0, i)),
        out_shape=jax.ShapeDtypeStruct((B, OUT), jnp.float32),
        compiler_params=pltpu.CompilerParams(
            dimension_semantics=("parallel",),
        ),
        name="linear_relu",
    )(flat, W_lin, b2)
    return out
